# baseline (speedup 1.0000x reference)
import functools

import jax
import jax.numpy as jnp
import numpy as np
from jax import lax
from jax.experimental import pallas as pl
from jax.experimental.pallas import tpu as pltpu
from jax.experimental.pallas import tpu_sc as plsc

_NC = 2
_NS = 16
_NW = _NC * _NS
_L = 16

_SEQ = 16
_CH = 8


def _mm_body(a_ref, w_ref, o_ref):
    o_ref[...] = jnp.dot(
        a_ref[...], w_ref[...], preferred_element_type=jnp.float32
    ).astype(jnp.bfloat16)


def _matmul_tc(a, w, block_rows):
    m, k = a.shape
    _, n = w.shape
    return pl.pallas_call(
        _mm_body,
        grid=(m // block_rows,),
        in_specs=[
            pl.BlockSpec((block_rows, k), lambda i: (i, 0)),
            pl.BlockSpec((k, n), lambda i: (0, 0)),
        ],
        out_specs=pl.BlockSpec((block_rows, n), lambda i: (i, 0)),
        out_shape=jax.ShapeDtypeStruct((m, n), jnp.bfloat16),
    )(a, w)


def _interleave_perm(c):
    p = np.arange(c)
    return (p // 32) * 32 + (p % 2) * _L + (p % 32) // 2


def _make_gather_reduce(np_nodes, c, apply_elu):
    pt = np_nodes // _NW
    nchunk = pt // _CH
    rows = _CH * _SEQ
    cvec = c // _L
    mask = jnp.uint32(0xFFFF0000).astype(jnp.int32)

    mesh = plsc.VectorSubcoreMesh(
        core_axis_name="c", subcore_axis_name="s", num_cores=_NC, num_subcores=_NS
    )

    @functools.partial(
        pl.kernel,
        out_type=jax.ShapeDtypeStruct((np_nodes, c), jnp.float32),
        mesh=mesh,
        compiler_params=pltpu.CompilerParams(use_tc_tiling_on_sc=False),
        scratch_types=[
            pltpu.VMEM((pt * _SEQ,), jnp.int32),
            pltpu.VMEM((rows, c // 2), jnp.int32),
            pltpu.VMEM((rows, c // 2), jnp.int32),
            pltpu.VMEM((_CH, c), jnp.float32),
            pltpu.VMEM((_CH, c), jnp.float32),
            pltpu.VMEM((c,), jnp.float32),
            pltpu.SemaphoreType.DMA,
            pltpu.SemaphoreType.DMA,
            pltpu.SemaphoreType.DMA,
            pltpu.SemaphoreType.DMA,
        ],
    )
    def body(table_hbm, idx_hbm, bias_hbm, out_hbm,
             idxv, gbuf0, gbuf1, obuf0, obuf1, biasv,
             gsem0, gsem1, osem0, osem1):
        wid = lax.axis_index("s") * _NC + lax.axis_index("c")
        base = wid * pt

        pltpu.sync_copy(idx_hbm.at[pl.ds(base * _SEQ, pt * _SEQ)], idxv)
        pltpu.sync_copy(bias_hbm, biasv)

        lane = lax.broadcasted_iota(jnp.int32, (_L,), 0)

        @pl.loop(0, pt * _SEQ // _L, unroll=4)
        def _flatten(i):
            off = pl.multiple_of(i * _L, _L)
            v = idxv[pl.ds(off, _L)]
            idxv[pl.ds(off, _L)] = v * _SEQ + lane

        bias_vecs = [biasv[pl.ds(j * _L, _L)] for j in range(cvec)]

        def fire(g, gbuf, gsem):
            roff = pl.multiple_of(g * rows, rows)
            pltpu.async_copy(table_hbm.at[idxv.at[pl.ds(roff, rows)]], gbuf, gsem)

        def reduce_chunk(g, gbuf, gsem, obuf, osem):
            pltpu.make_async_copy(table_hbm.at[pl.ds(0, rows)], gbuf, gsem).wait()

            @pl.when(g >= 2)
            def _():
                pltpu.make_async_copy(out_hbm.at[pl.ds(0, _CH)], obuf, osem).wait()

            for nloc in range(_CH):
                acc = list(bias_vecs)
                for s in range(_SEQ):
                    r = nloc * _SEQ + s
                    for b in range(c // 32):
                        w = gbuf[r, pl.ds(b * _L, _L)]
                        lo = lax.bitcast_convert_type(
                            lax.shift_left(w, 16), jnp.float32)
                        hi = lax.bitcast_convert_type(w & mask, jnp.float32)
                        acc[2 * b] = acc[2 * b] + lo
                        acc[2 * b + 1] = acc[2 * b + 1] + hi
                for j in range(cvec):
                    v = acc[j]
                    if apply_elu:
                        v = jnp.where(v > 0.0, v, jnp.exp(v) - 1.0)
                    obuf[nloc, pl.ds(j * _L, _L)] = v
            @pl.when(g + 2 < nchunk)
            def _():
                fire(g + 2, gbuf, gsem)
            pltpu.async_copy(obuf, out_hbm.at[pl.ds(base + g * _CH, _CH)], osem)

        fire(0, gbuf0, gsem0)
        fire(1, gbuf1, gsem1)

        @pl.loop(0, nchunk // 2)
        def _pair(h):
            g0 = pl.multiple_of(h * 2, 2)
            reduce_chunk(g0, gbuf0, gsem0, obuf0, osem0)
            reduce_chunk(g0 + 1, gbuf1, gsem1, obuf1, osem1)

        pltpu.make_async_copy(out_hbm.at[pl.ds(0, _CH)], obuf0, osem0).wait()
        pltpu.make_async_copy(out_hbm.at[pl.ds(0, _CH)], obuf1, osem1).wait()

    return body


def kernel(x, spiral_indices, W1, b1, W2, b2):
    n = x.shape[0]
    c0 = x.shape[1]
    c1 = W1.shape[1]
    c2 = W2.shape[1]

    blk = 1024
    np_nodes = ((n + blk - 1) // blk) * blk

    h0 = jnp.pad(x[:, :, 0], ((0, np_nodes - n), (0, 0)))
    idx_flat = jnp.pad(spiral_indices, ((0, np_nodes - n), (0, 0))).reshape(-1)

    p1 = _interleave_perm(c1)
    p2 = _interleave_perm(c2)
    w1c = W1.reshape(_SEQ, c0, c1).transpose(1, 0, 2)[:, :, p1].reshape(c0, _SEQ * c1)
    w2c = W2.reshape(_SEQ, c1, c2).transpose(1, 0, 2)[:, :, p2].reshape(c1, _SEQ * c2)

    def as_i32_table(y, c):
        return lax.bitcast_convert_type(
            y.reshape(np_nodes * _SEQ, c // 2, 2), jnp.int32)

    y1 = _matmul_tc(h0, w1c, blk)
    g1 = _make_gather_reduce(np_nodes, c1, True)
    h1 = g1(as_i32_table(y1, c1), idx_flat, b1)

    y2 = _matmul_tc(h1, w2c, blk)
    g2 = _make_gather_reduce(np_nodes, c2, False)
    out = g2(as_i32_table(y2, c2), idx_flat, b2)

    return out[:n, :, None]


# speedup vs baseline: 82.2285x; 82.2285x over previous
import functools

import jax
import jax.numpy as jnp
import numpy as np
from jax import lax
from jax.experimental import pallas as pl
from jax.experimental.pallas import tpu as pltpu
from jax.experimental.pallas import tpu_sc as plsc

_NC = 2
_NS = 16
_NW = _NC * _NS
_L = 16

_SEQ = 16
_CH = 8

_HIMASK = np.int32(-65536)
_RND = np.int32(0x8000)


def _mm_pack_body(a_ref, w_ref, o_ref):
    half = o_ref.shape[1]
    p = jnp.dot(a_ref[...], w_ref[...], preferred_element_type=jnp.float32)
    lo = lax.bitcast_convert_type(p[:, :half], jnp.int32)
    hi = lax.bitcast_convert_type(p[:, half:], jnp.int32)
    o_ref[...] = lax.shift_right_logical(lo + _RND, 16) | ((hi + _RND) & _HIMASK)


def _matmul_pack_tc(a, w, block_rows):
    m, k = a.shape
    _, n2 = w.shape
    half = n2 // 2
    return pl.pallas_call(
        _mm_pack_body,
        grid=(m // block_rows,),
        in_specs=[
            pl.BlockSpec((block_rows, k), lambda i: (i, 0)),
            pl.BlockSpec((k, n2), lambda i: (0, 0)),
        ],
        out_specs=pl.BlockSpec((block_rows, half), lambda i: (i, 0)),
        out_shape=jax.ShapeDtypeStruct((m, half), jnp.int32),
    )(a, w)


def _make_gather_reduce(np_nodes, c, apply_elu):
    pt = np_nodes // _NW
    nchunk = pt // _CH
    rows = _CH * _SEQ
    cvec = c // _L
    cw = c // 2

    mesh = plsc.VectorSubcoreMesh(
        core_axis_name="c", subcore_axis_name="s", num_cores=_NC, num_subcores=_NS
    )

    @functools.partial(
        pl.kernel,
        out_type=jax.ShapeDtypeStruct((np_nodes, c), jnp.float32),
        mesh=mesh,
        compiler_params=pltpu.CompilerParams(use_tc_tiling_on_sc=False),
        scratch_types=[
            pltpu.VMEM((pt * _SEQ,), jnp.int32),
            pltpu.VMEM((rows, cw), jnp.int32),
            pltpu.VMEM((rows, cw), jnp.int32),
            pltpu.VMEM((_CH, c), jnp.float32),
            pltpu.VMEM((_CH, c), jnp.float32),
            pltpu.VMEM((c,), jnp.float32),
            pltpu.SemaphoreType.DMA,
            pltpu.SemaphoreType.DMA,
            pltpu.SemaphoreType.DMA,
            pltpu.SemaphoreType.DMA,
        ],
    )
    def body(table_hbm, idx_hbm, bias_hbm, out_hbm,
             idxv, gbuf0, gbuf1, obuf0, obuf1, biasv,
             gsem0, gsem1, osem0, osem1):
        wid = lax.axis_index("s") * _NC + lax.axis_index("c")
        base = wid * pt

        pltpu.sync_copy(idx_hbm.at[pl.ds(base * _SEQ, pt * _SEQ)], idxv)
        pltpu.sync_copy(bias_hbm, biasv)

        lane = lax.broadcasted_iota(jnp.int32, (_L,), 0)

        @pl.loop(0, pt * _SEQ // _L, unroll=4)
        def _flatten(i):
            off = pl.multiple_of(i * _L, _L)
            v = idxv[pl.ds(off, _L)]
            idxv[pl.ds(off, _L)] = v * _SEQ + lane

        bias_vecs = [biasv[pl.ds(j * _L, _L)] for j in range(cvec)]

        def fire(g, gbuf, gsem):
            roff = pl.multiple_of(g * rows, rows)
            pltpu.async_copy(table_hbm.at[idxv.at[pl.ds(roff, rows)]], gbuf, gsem)

        def reduce_chunk(g, gbuf, gsem, obuf, osem):
            pltpu.make_async_copy(table_hbm.at[pl.ds(0, rows)], gbuf, gsem).wait()

            @pl.when(g >= 2)
            def _():
                pltpu.make_async_copy(out_hbm.at[pl.ds(0, _CH)], obuf, osem).wait()

            for nloc in range(_CH):
                acc = list(bias_vecs)
                for s in range(_SEQ):
                    r = nloc * _SEQ + s
                    for b in range(cw // _L):
                        w = gbuf[r, pl.ds(b * _L, _L)]
                        lo = lax.bitcast_convert_type(
                            lax.shift_left(w, 16), jnp.float32)
                        hi = lax.bitcast_convert_type(w & _HIMASK, jnp.float32)
                        acc[2 * b] = acc[2 * b] + lo
                        acc[2 * b + 1] = acc[2 * b + 1] + hi
                for j in range(cvec):
                    v = acc[j]
                    if apply_elu:
                        v = jnp.where(v > 0.0, v, jnp.exp(v) - 1.0)
                    obuf[nloc, pl.ds(j * _L, _L)] = v
            @pl.when(g + 2 < nchunk)
            def _():
                fire(g + 2, gbuf, gsem)
            pltpu.async_copy(obuf, out_hbm.at[pl.ds(base + g * _CH, _CH)], osem)

        fire(0, gbuf0, gsem0)
        fire(1, gbuf1, gsem1)

        @pl.loop(0, nchunk // 2)
        def _pair(h):
            g0 = pl.multiple_of(h * 2, 2)
            reduce_chunk(g0, gbuf0, gsem0, obuf0, osem0)
            reduce_chunk(g0 + 1, gbuf1, gsem1, obuf1, osem1)

        pltpu.make_async_copy(out_hbm.at[pl.ds(0, _CH)], obuf0, osem0).wait()
        pltpu.make_async_copy(out_hbm.at[pl.ds(0, _CH)], obuf1, osem1).wait()

    return body


def _split_weights(W, c_in, c):
    wt = W.reshape(_SEQ, c_in, c).transpose(1, 0, 2)
    j = np.arange(c // 2)
    lo_ch = (j // _L) * 2 * _L + (j % _L)
    wa = wt[:, :, lo_ch].reshape(c_in, _SEQ * c // 2)
    wb = wt[:, :, lo_ch + _L].reshape(c_in, _SEQ * c // 2)
    return jnp.concatenate([wa, wb], axis=1)


def kernel(x, spiral_indices, W1, b1, W2, b2):
    n = x.shape[0]
    c0 = x.shape[1]
    c1 = W1.shape[1]
    c2 = W2.shape[1]

    blk = 1024
    np_nodes = ((n + blk - 1) // blk) * blk

    h0 = jnp.pad(x[:, :, 0], ((0, np_nodes - n), (0, 0)))
    idx_flat = jnp.pad(spiral_indices, ((0, np_nodes - n), (0, 0))).reshape(-1)

    w1c = _split_weights(W1, c0, c1)
    w2c = _split_weights(W2, c1, c2)

    y1 = _matmul_pack_tc(h0, w1c, blk)
    g1 = _make_gather_reduce(np_nodes, c1, True)
    h1 = g1(y1.reshape(np_nodes * _SEQ, c1 // 2), idx_flat, b1)

    y2 = _matmul_pack_tc(h1, w2c, blk)
    g2 = _make_gather_reduce(np_nodes, c2, False)
    out = g2(y2.reshape(np_nodes * _SEQ, c2 // 2), idx_flat, b2)

    return out[:n, :, None]


# speedup vs baseline: 100.6761x; 1.2243x over previous
import functools

import jax
import jax.numpy as jnp
import numpy as np
from jax import lax
from jax.experimental import pallas as pl
from jax.experimental.pallas import tpu as pltpu
from jax.experimental.pallas import tpu_sc as plsc

_NC = 2
_NS = 16
_NW = _NC * _NS
_L = 16

_SEQ = 16
_CH = 8

_HIMASK = np.int32(-65536)
_RND = np.int32(0x8000)


def _mm_pack_body(a_ref, w_ref, o_ref):
    blk = a_ref.shape[0]
    half = w_ref.shape[1] // 2
    p = jnp.dot(a_ref[...], w_ref[...], preferred_element_type=jnp.float32)
    lo = lax.bitcast_convert_type(p[:, :half], jnp.int32)
    hi = lax.bitcast_convert_type(p[:, half:], jnp.int32)
    packed = lax.shift_right_logical(lo + _RND, 16) | ((hi + _RND) & _HIMASK)
    o_ref[...] = packed.reshape(blk * half // 128, 128)


def _matmul_pack_tc(a, w, block_rows):
    m, k = a.shape
    _, n2 = w.shape
    half = n2 // 2
    rr = half // 128
    return pl.pallas_call(
        _mm_pack_body,
        grid=(m // block_rows,),
        in_specs=[
            pl.BlockSpec((block_rows, k), lambda i: (i, 0)),
            pl.BlockSpec((k, n2), lambda i: (0, 0)),
        ],
        out_specs=pl.BlockSpec((block_rows * rr, 128), lambda i: (i, 0)),
        out_shape=jax.ShapeDtypeStruct((m * rr, 128), jnp.int32),
    )(a, w)


def _make_gather_reduce(np_nodes, c, apply_elu):
    pt = np_nodes // _NW
    nchunk = pt // _CH
    rows = _CH * _SEQ
    cvec = c // _L
    cw = c // 2

    mesh = plsc.VectorSubcoreMesh(
        core_axis_name="c", subcore_axis_name="s", num_cores=_NC, num_subcores=_NS
    )

    @functools.partial(
        pl.kernel,
        out_type=jax.ShapeDtypeStruct((np_nodes, c), jnp.float32),
        mesh=mesh,
        compiler_params=pltpu.CompilerParams(use_tc_tiling_on_sc=False),
        scratch_types=[
            pltpu.VMEM((pt * _SEQ,), jnp.int32),
            pltpu.VMEM((rows, cw), jnp.int32),
            pltpu.VMEM((rows, cw), jnp.int32),
            pltpu.VMEM((_CH, c), jnp.float32),
            pltpu.VMEM((_CH, c), jnp.float32),
            pltpu.VMEM((c,), jnp.float32),
            pltpu.SemaphoreType.DMA,
            pltpu.SemaphoreType.DMA,
            pltpu.SemaphoreType.DMA,
            pltpu.SemaphoreType.DMA,
        ],
    )
    def body(table_hbm, idx_hbm, bias_hbm, out_hbm,
             idxv, gbuf0, gbuf1, obuf0, obuf1, biasv,
             gsem0, gsem1, osem0, osem1):
        wid = lax.axis_index("s") * _NC + lax.axis_index("c")
        base = wid * pt

        pltpu.sync_copy(idx_hbm.at[pl.ds(base * _SEQ, pt * _SEQ)], idxv)
        pltpu.sync_copy(bias_hbm, biasv)

        lane = lax.broadcasted_iota(jnp.int32, (_L,), 0)

        @pl.loop(0, pt * _SEQ // _L, unroll=4)
        def _flatten(i):
            off = pl.multiple_of(i * _L, _L)
            v = idxv[pl.ds(off, _L)]
            idxv[pl.ds(off, _L)] = v * _SEQ + lane

        bias_vecs = [biasv[pl.ds(j * _L, _L)] for j in range(cvec)]

        def fire(g, gbuf, gsem):
            roff = pl.multiple_of(g * rows, rows)
            pltpu.async_copy(table_hbm.at[idxv.at[pl.ds(roff, rows)]], gbuf, gsem)

        def reduce_chunk(g, gbuf, gsem, obuf, osem):
            pltpu.make_async_copy(table_hbm.at[pl.ds(0, rows)], gbuf, gsem).wait()

            @pl.when(g >= 2)
            def _():
                pltpu.make_async_copy(out_hbm.at[pl.ds(0, _CH)], obuf, osem).wait()

            for nloc in range(_CH):
                acc = list(bias_vecs)
                for s in range(_SEQ):
                    r = nloc * _SEQ + s
                    for b in range(cw // _L):
                        w = gbuf[r, pl.ds(b * _L, _L)]
                        lo = lax.bitcast_convert_type(
                            lax.shift_left(w, 16), jnp.float32)
                        hi = lax.bitcast_convert_type(w & _HIMASK, jnp.float32)
                        acc[2 * b] = acc[2 * b] + lo
                        acc[2 * b + 1] = acc[2 * b + 1] + hi
                for j in range(cvec):
                    v = acc[j]
                    if apply_elu:
                        v = jnp.where(v > 0.0, v, jnp.exp(v) - 1.0)
                    obuf[nloc, pl.ds(j * _L, _L)] = v
            @pl.when(g + 2 < nchunk)
            def _():
                fire(g + 2, gbuf, gsem)
            pltpu.async_copy(obuf, out_hbm.at[pl.ds(base + g * _CH, _CH)], osem)

        fire(0, gbuf0, gsem0)
        fire(1, gbuf1, gsem1)

        @pl.loop(0, nchunk // 2)
        def _pair(h):
            g0 = pl.multiple_of(h * 2, 2)
            reduce_chunk(g0, gbuf0, gsem0, obuf0, osem0)
            reduce_chunk(g0 + 1, gbuf1, gsem1, obuf1, osem1)

        pltpu.make_async_copy(out_hbm.at[pl.ds(0, _CH)], obuf0, osem0).wait()
        pltpu.make_async_copy(out_hbm.at[pl.ds(0, _CH)], obuf1, osem1).wait()

    return body


def _split_weights(W, c_in, c):
    wt = W.reshape(_SEQ, c_in, c).transpose(1, 0, 2)
    j = np.arange(c // 2)
    lo_ch = (j // _L) * 2 * _L + (j % _L)
    wa = wt[:, :, lo_ch].reshape(c_in, _SEQ * c // 2)
    wb = wt[:, :, lo_ch + _L].reshape(c_in, _SEQ * c // 2)
    return jnp.concatenate([wa, wb], axis=1)


def kernel(x, spiral_indices, W1, b1, W2, b2):
    n = x.shape[0]
    c0 = x.shape[1]
    c1 = W1.shape[1]
    c2 = W2.shape[1]

    blk = 1024
    np_nodes = ((n + blk - 1) // blk) * blk

    h0 = jnp.pad(x[:, :, 0], ((0, np_nodes - n), (0, 0)))
    idx_flat = jnp.pad(spiral_indices, ((0, np_nodes - n), (0, 0))).reshape(-1)

    w1c = _split_weights(W1, c0, c1)
    w2c = _split_weights(W2, c1, c2)

    y1 = _matmul_pack_tc(h0, w1c, blk)
    g1 = _make_gather_reduce(np_nodes, c1, True)
    h1 = g1(y1.reshape(np_nodes * _SEQ, c1 // 2), idx_flat, b1)

    y2 = _matmul_pack_tc(h1, w2c, blk)
    g2 = _make_gather_reduce(np_nodes, c2, False)
    out = g2(y2.reshape(np_nodes * _SEQ, c2 // 2), idx_flat, b2)

    return out[:n, :, None]


# speedup vs baseline: 120.5485x; 1.1974x over previous
import functools

import jax
import jax.numpy as jnp
import numpy as np
from jax import lax
from jax.experimental import pallas as pl
from jax.experimental.pallas import tpu as pltpu
from jax.experimental.pallas import tpu_sc as plsc

_NC = 2
_NS = 16
_NW = _NC * _NS
_L = 16

_SEQ = 16
_CH = 8

_HIMASK = np.int32(-65536)
_RND = np.int32(0x8000)


def _pack_words(a, b):
    lo = lax.bitcast_convert_type(a, jnp.int32)
    hi = lax.bitcast_convert_type(b, jnp.int32)
    return lax.shift_right_logical(lo + _RND, 16) | ((hi + _RND) & _HIMASK)


def _mm1_body(a_ref, w_ref, o_ref):
    blk = a_ref.shape[0]
    half = w_ref.shape[1] // 2
    p = jnp.dot(a_ref[...], w_ref[...], preferred_element_type=jnp.float32)
    packed = _pack_words(p[:, :half], p[:, half:])
    o_ref[...] = packed.reshape(blk * half // 128, 128)


def _mm2_body(a_ref, w_ref, o_ref):
    blk = a_ref.shape[0]
    q = w_ref.shape[1] // 4
    p = jnp.dot(a_ref[...], w_ref[...], preferred_element_type=jnp.float32)
    even = _pack_words(p[:, :q], p[:, q:2 * q])
    odd = _pack_words(p[:, 2 * q:3 * q], p[:, 3 * q:])
    packed = jnp.concatenate([even, odd], axis=1)
    o_ref[...] = packed.reshape(blk * q // 64, 128)


def _matmul_pack_tc(body, a, w, block_rows, out_rows_per_in, grid):
    m, _ = a.shape
    _, n2 = w.shape
    rr = out_rows_per_in
    return pl.pallas_call(
        body,
        grid=(grid,),
        in_specs=[
            pl.BlockSpec((block_rows, a.shape[1]), lambda i: (i, 0)),
            pl.BlockSpec((a.shape[1], n2), lambda i: (0, 0)),
        ],
        out_specs=pl.BlockSpec((block_rows * rr, 128), lambda i: (i, 0)),
        out_shape=jax.ShapeDtypeStruct((grid * block_rows * rr, 128), jnp.int32),
    )(a, w)


def _make_gather_reduce(n, c, apply_elu, pack_out):
    pt = -(-n // _NW)
    pt = -(-pt // _CH) * _CH
    nchunk = pt // _CH
    nchunk += nchunk % 2
    pt = nchunk * _CH
    rows = _CH * _SEQ
    cvec = c // _L
    cw = c // 2
    npr = 128 // c if pack_out else 1
    out_shape = (n * c // 128, 128) if pack_out else (n, c)
    och = _CH // npr

    mesh = plsc.VectorSubcoreMesh(
        core_axis_name="c", subcore_axis_name="s", num_cores=_NC, num_subcores=_NS
    )

    @functools.partial(
        pl.kernel,
        out_type=jax.ShapeDtypeStruct(out_shape, jnp.float32),
        mesh=mesh,
        compiler_params=pltpu.CompilerParams(use_tc_tiling_on_sc=False),
        scratch_types=[
            pltpu.VMEM((pt, _SEQ), jnp.int32),
            pltpu.VMEM((pt * _SEQ,), jnp.int32),
            pltpu.VMEM((rows, cw), jnp.int32),
            pltpu.VMEM((rows, cw), jnp.int32),
            pltpu.VMEM((och, 128 if pack_out else c), jnp.float32),
            pltpu.VMEM((och, 128 if pack_out else c), jnp.float32),
            pltpu.VMEM((c,), jnp.float32),
            pltpu.SemaphoreType.DMA,
            pltpu.SemaphoreType.DMA,
            pltpu.SemaphoreType.DMA,
            pltpu.SemaphoreType.DMA,
        ],
    )
    def body(table_hbm, idx_hbm, bias_hbm, out_hbm,
             idx2d, idxv, gbuf0, gbuf1, obuf0, obuf1, biasv,
             gsem0, gsem1, osem0, osem1):
        wid = lax.axis_index("s") * _NC + lax.axis_index("c")
        base = jnp.minimum(wid * pt, n - pt)

        pltpu.sync_copy(idx_hbm.at[pl.ds(base, pt), :], idx2d)
        pltpu.sync_copy(bias_hbm, biasv)

        lane = lax.broadcasted_iota(jnp.int32, (_L,), 0)

        @pl.loop(0, pt, unroll=4)
        def _flatten(i):
            v = idx2d[i, :]
            idxv[pl.ds(pl.multiple_of(i * _SEQ, _SEQ), _SEQ)] = v * _SEQ + lane

        bias_vecs = [biasv[pl.ds(j * _L, _L)] for j in range(cvec)]

        def fire(g, gbuf, gsem):
            roff = pl.multiple_of(g * rows, rows)
            pltpu.async_copy(table_hbm.at[idxv.at[pl.ds(roff, rows)]], gbuf, gsem)

        def reduce_chunk(g, gbuf, gsem, obuf, osem):
            pltpu.make_async_copy(table_hbm.at[pl.ds(0, rows)], gbuf, gsem).wait()

            @pl.when(g >= 2)
            def _():
                pltpu.make_async_copy(out_hbm.at[pl.ds(0, och)], obuf, osem).wait()

            for nloc in range(_CH):
                acc = list(bias_vecs)
                for s in range(_SEQ):
                    r = nloc * _SEQ + s
                    for b in range(cw // _L):
                        w = gbuf[r, pl.ds(b * _L, _L)]
                        lo = lax.bitcast_convert_type(
                            lax.shift_left(w, 16), jnp.float32)
                        hi = lax.bitcast_convert_type(w & _HIMASK, jnp.float32)
                        acc[2 * b] = acc[2 * b] + lo
                        acc[2 * b + 1] = acc[2 * b + 1] + hi
                for j in range(cvec):
                    v = acc[j]
                    if apply_elu:
                        v = jnp.where(v > 0.0, v, jnp.exp(v) - 1.0)
                    obuf[nloc // npr, pl.ds((nloc % npr) * c + j * _L, _L)] = v
            @pl.when(g + 2 < nchunk)
            def _():
                fire(g + 2, gbuf, gsem)
            pltpu.async_copy(
                obuf, out_hbm.at[pl.ds((base + g * _CH) // npr, och)], osem)

        fire(0, gbuf0, gsem0)
        fire(1, gbuf1, gsem1)

        @pl.loop(0, nchunk // 2)
        def _pair(h):
            g0 = pl.multiple_of(h * 2, 2)
            reduce_chunk(g0, gbuf0, gsem0, obuf0, osem0)
            reduce_chunk(g0 + 1, gbuf1, gsem1, obuf1, osem1)

        pltpu.make_async_copy(out_hbm.at[pl.ds(0, och)], obuf0, osem0).wait()
        pltpu.make_async_copy(out_hbm.at[pl.ds(0, och)], obuf1, osem1).wait()

    return body


def _split_weights(W, c_in, c):
    wt = W.reshape(_SEQ, c_in, c).transpose(1, 0, 2)
    j = np.arange(c // 2)
    lo_ch = (j // _L) * 2 * _L + (j % _L)
    wa = wt[:, :, lo_ch].reshape(c_in, _SEQ * c // 2)
    wb = wt[:, :, lo_ch + _L].reshape(c_in, _SEQ * c // 2)
    return jnp.concatenate([wa, wb], axis=1)


def kernel(x, spiral_indices, W1, b1, W2, b2):
    n = x.shape[0]
    c0 = x.shape[1]
    c1 = W1.shape[1]
    c2 = W2.shape[1]

    blk = 1024
    grid1 = -(-n // blk)
    h0 = x[:, :, 0]

    w1c = _split_weights(W1, c0, c1)
    w2c = _split_weights(W2, c1, c2)
    z = jnp.zeros_like(w2c)
    w2d = jnp.concatenate([
        jnp.concatenate([w2c, z], axis=1),
        jnp.concatenate([z, w2c], axis=1),
    ], axis=0)

    rr1 = _SEQ * c1 // 2 // 128
    y1 = _matmul_pack_tc(_mm1_body, h0, w1c, blk, rr1, grid1)
    g1 = _make_gather_reduce(n, c1, True, True)
    h1p = g1(y1.reshape(grid1 * blk * _SEQ, c1 // 2), spiral_indices, b1)

    blk2 = 1000
    rr2 = _SEQ * c2 // 128
    y2 = _matmul_pack_tc(_mm2_body, h1p, w2d, blk2, rr2, n // 2 // blk2)
    g2 = _make_gather_reduce(n, c2, False, False)
    out = g2(y2.reshape(n * _SEQ, c2 // 2), spiral_indices, b2)

    return out[:, :, None]


# speedup vs baseline: 129.1940x; 1.0717x over previous
import functools

import jax
import jax.numpy as jnp
import numpy as np
from jax import lax
from jax.experimental import pallas as pl
from jax.experimental.pallas import tpu as pltpu
from jax.experimental.pallas import tpu_sc as plsc

_NC = 2
_NS = 16
_NW = _NC * _NS
_L = 16

_SEQ = 16
_CH = 8

_HIMASK = np.int32(-65536)
_RND = np.int32(0x8000)


def _pack_words(a, b):
    lo = lax.bitcast_convert_type(a, jnp.int32)
    hi = lax.bitcast_convert_type(b, jnp.int32)
    return lax.shift_right_logical(lo + _RND, 16) | ((hi + _RND) & _HIMASK)


def _mm1_body(a_ref, w_ref, o_ref):
    blk = a_ref.shape[0]
    half = w_ref.shape[1] // 2
    a = a_ref[...].reshape(blk, a_ref.shape[1])
    p = jnp.dot(a, w_ref[...], preferred_element_type=jnp.float32)
    packed = _pack_words(p[:, :half], p[:, half:])
    o_ref[...] = packed.reshape(blk * half // 128, 128)


def _mm2_body(a_ref, w_ref, o_ref):
    blk = a_ref.shape[0]
    q = w_ref.shape[1] // 4
    p = jnp.dot(a_ref[...], w_ref[...], preferred_element_type=jnp.float32)
    even = _pack_words(p[:, :q], p[:, q:2 * q])
    odd = _pack_words(p[:, 2 * q:3 * q], p[:, 3 * q:])
    packed = jnp.concatenate([even, odd], axis=1)
    o_ref[...] = packed.reshape(blk * q // 64, 128)


def _matmul_pack_tc(body, a, w, block_rows, out_rows_per_in, grid):
    _, n2 = w.shape
    rr = out_rows_per_in
    a_blk = (block_rows,) + a.shape[1:]
    a_map = (lambda i: (i, 0, 0)) if a.ndim == 3 else (lambda i: (i, 0))
    return pl.pallas_call(
        body,
        grid=(grid,),
        in_specs=[
            pl.BlockSpec(a_blk, a_map),
            pl.BlockSpec((w.shape[0], n2), lambda i: (0, 0)),
        ],
        out_specs=pl.BlockSpec((block_rows * rr, 128), lambda i: (i, 0)),
        out_shape=jax.ShapeDtypeStruct((grid * block_rows * rr, 128), jnp.int32),
    )(a, w)


def _make_gather_reduce(n, c, apply_elu, pack_out, chn):
    pt = -(-n // _NW)
    pt = -(-pt // (2 * chn)) * (2 * chn)
    nchunk = pt // chn
    rows = _CH * _SEQ
    ndma = chn // _CH
    cvec = c // _L
    cw = c // 2
    npr = 128 // c if pack_out else 1
    out_shape = (n * c // 128, 128) if pack_out else (n, c)
    och = chn // npr

    mesh = plsc.VectorSubcoreMesh(
        core_axis_name="c", subcore_axis_name="s", num_cores=_NC, num_subcores=_NS
    )

    @functools.partial(
        pl.kernel,
        out_type=jax.ShapeDtypeStruct(out_shape, jnp.float32),
        mesh=mesh,
        compiler_params=pltpu.CompilerParams(use_tc_tiling_on_sc=False),
        scratch_types=[
            pltpu.VMEM((pt, _SEQ), jnp.int32),
            pltpu.VMEM((pt * _SEQ,), jnp.int32),
            pltpu.VMEM((chn * _SEQ, cw), jnp.int32),
            pltpu.VMEM((chn * _SEQ, cw), jnp.int32),
            pltpu.VMEM((och, 128 if pack_out else c), jnp.float32),
            pltpu.VMEM((och, 128 if pack_out else c), jnp.float32),
            pltpu.VMEM((c,), jnp.float32),
            pltpu.SemaphoreType.DMA,
            pltpu.SemaphoreType.DMA,
            pltpu.SemaphoreType.DMA,
            pltpu.SemaphoreType.DMA,
        ],
    )
    def body(table_hbm, idx_hbm, bias_hbm, out_hbm,
             idx2d, idxv, gbuf0, gbuf1, obuf0, obuf1, biasv,
             gsem0, gsem1, osem0, osem1):
        wid = lax.axis_index("s") * _NC + lax.axis_index("c")
        base = jnp.minimum(wid * pt, n - pt)

        pltpu.sync_copy(idx_hbm.at[pl.ds(base, pt), :], idx2d)
        pltpu.sync_copy(bias_hbm, biasv)

        lane = lax.broadcasted_iota(jnp.int32, (_L,), 0)

        @pl.loop(0, pt, unroll=4)
        def _flatten(i):
            v = idx2d[i, :]
            idxv[pl.ds(pl.multiple_of(i * _SEQ, _SEQ), _SEQ)] = v * _SEQ + lane

        bias_vecs = [biasv[pl.ds(j * _L, _L)] for j in range(cvec)]

        def fire(g, gbuf, gsem):
            for d in range(ndma):
                roff = pl.multiple_of(g * chn * _SEQ + d * rows, rows)
                pltpu.async_copy(
                    table_hbm.at[idxv.at[pl.ds(roff, rows)]],
                    gbuf.at[pl.ds(d * rows, rows)], gsem)

        def reduce_chunk(g, gbuf, gsem, obuf, osem):
            pltpu.make_async_copy(
                table_hbm.at[pl.ds(0, chn * _SEQ)], gbuf, gsem).wait()

            @pl.when(g >= 2)
            def _():
                pltpu.make_async_copy(out_hbm.at[pl.ds(0, och)], obuf, osem).wait()

            for nloc in range(chn):
                acc = list(bias_vecs)
                for s in range(_SEQ):
                    r = nloc * _SEQ + s
                    for b in range(cw // _L):
                        w = gbuf[r, pl.ds(b * _L, _L)]
                        lo = lax.bitcast_convert_type(
                            lax.shift_left(w, 16), jnp.float32)
                        hi = lax.bitcast_convert_type(w & _HIMASK, jnp.float32)
                        acc[2 * b] = acc[2 * b] + lo
                        acc[2 * b + 1] = acc[2 * b + 1] + hi
                for j in range(cvec):
                    v = acc[j]
                    if apply_elu:
                        v = jnp.where(v > 0.0, v, jnp.exp(v) - 1.0)
                    obuf[nloc // npr, pl.ds((nloc % npr) * c + j * _L, _L)] = v
            @pl.when(g + 2 < nchunk)
            def _():
                fire(g + 2, gbuf, gsem)
            pltpu.async_copy(
                obuf, out_hbm.at[pl.ds((base + g * chn) // npr, och)], osem)

        fire(0, gbuf0, gsem0)
        fire(1, gbuf1, gsem1)

        @pl.loop(0, nchunk // 2)
        def _pair(h):
            g0 = pl.multiple_of(h * 2, 2)
            reduce_chunk(g0, gbuf0, gsem0, obuf0, osem0)
            reduce_chunk(g0 + 1, gbuf1, gsem1, obuf1, osem1)

        pltpu.make_async_copy(out_hbm.at[pl.ds(0, och)], obuf0, osem0).wait()
        pltpu.make_async_copy(out_hbm.at[pl.ds(0, och)], obuf1, osem1).wait()

    return body


def _split_weights(W, c_in, c):
    wt = W.reshape(_SEQ, c_in, c).transpose(1, 0, 2)
    j = np.arange(c // 2)
    lo_ch = (j // _L) * 2 * _L + (j % _L)
    wa = wt[:, :, lo_ch].reshape(c_in, _SEQ * c // 2)
    wb = wt[:, :, lo_ch + _L].reshape(c_in, _SEQ * c // 2)
    return jnp.concatenate([wa, wb], axis=1)


def kernel(x, spiral_indices, W1, b1, W2, b2):
    n = x.shape[0]
    c0 = x.shape[1]
    c1 = W1.shape[1]
    c2 = W2.shape[1]

    blk = 1024
    grid1 = -(-n // blk)

    w1c = _split_weights(W1, c0, c1)
    w2c = _split_weights(W2, c1, c2)
    z = jnp.zeros_like(w2c)
    w2d = jnp.concatenate([
        jnp.concatenate([w2c, z], axis=1),
        jnp.concatenate([z, w2c], axis=1),
    ], axis=0)

    rr1 = _SEQ * c1 // 2 // 128
    y1 = _matmul_pack_tc(_mm1_body, x[:, :, 0], w1c, blk, rr1, grid1)
    g1 = _make_gather_reduce(n, c1, True, True, _CH)
    h1p = g1(y1.reshape(grid1 * blk * _SEQ, c1 // 2), spiral_indices, b1)

    blk2 = 1000
    rr2 = _SEQ * c2 // 128
    y2 = _matmul_pack_tc(_mm2_body, h1p, w2d, blk2, rr2, n // 2 // blk2)
    g2 = _make_gather_reduce(n, c2, False, True, 2 * _CH)
    out = g2(y2.reshape(n * _SEQ, c2 // 2), spiral_indices, b2)

    return out.reshape(n, c2)[:, :, None]


# speedup vs baseline: 136.5128x; 1.0566x over previous
import functools

import jax
import jax.numpy as jnp
import numpy as np
from jax import lax
from jax.experimental import pallas as pl
from jax.experimental.pallas import tpu as pltpu
from jax.experimental.pallas import tpu_sc as plsc

_NC = 2
_NS = 16
_NW = _NC * _NS
_L = 16

_SEQ = 16
_CH = 8

_HIMASK = np.int32(-65536)
_RND = np.int32(0x8000)


def _pack_words(a, b):
    lo = lax.bitcast_convert_type(a, jnp.int32)
    hi = lax.bitcast_convert_type(b, jnp.int32)
    return lax.shift_right_logical(lo + _RND, 16) | ((hi + _RND) & _HIMASK)


def _mm1_body(a_ref, w_ref, o_ref):
    blk = a_ref.shape[0]
    half = w_ref.shape[1] // 2
    a = a_ref[...].reshape(blk, a_ref.shape[1])
    p = jnp.dot(a, w_ref[...], preferred_element_type=jnp.float32)
    packed = _pack_words(p[:, :half], p[:, half:])
    o_ref[...] = packed.reshape(blk * half // 128, 128)


def _mm2_body(a_ref, w_ref, o_ref):
    blk = a_ref.shape[0]
    q = w_ref.shape[1] // 4
    p = jnp.dot(a_ref[...], w_ref[...], preferred_element_type=jnp.float32)
    even = _pack_words(p[:, :q], p[:, q:2 * q])
    odd = _pack_words(p[:, 2 * q:3 * q], p[:, 3 * q:])
    packed = jnp.concatenate([even, odd], axis=1)
    o_ref[...] = packed.reshape(blk * q // 64, 128)


def _matmul_pack_tc(body, a, w, block_rows, out_rows_per_in, grid):
    _, n2 = w.shape
    rr = out_rows_per_in
    a_blk = (block_rows,) + a.shape[1:]
    a_map = (lambda i: (i, 0, 0)) if a.ndim == 3 else (lambda i: (i, 0))
    return pl.pallas_call(
        body,
        grid=(grid,),
        in_specs=[
            pl.BlockSpec(a_blk, a_map),
            pl.BlockSpec((w.shape[0], n2), lambda i: (0, 0)),
        ],
        out_specs=pl.BlockSpec((block_rows * rr, 128), lambda i: (i, 0)),
        out_shape=jax.ShapeDtypeStruct((grid * block_rows * rr, 128), jnp.int32),
    )(a, w)


def _make_gather_reduce(n, c, apply_elu, pack_out, chn):
    pt = -(-n // _NW)
    pt = -(-pt // (2 * chn)) * (2 * chn)
    nchunk = pt // chn
    rows = _CH * _SEQ
    ndma = chn // _CH
    cvec = c // _L
    cw = c // 2
    npr = 128 // c if pack_out else 1
    out_shape = (n * c // 128, 128) if pack_out else (n, c)
    och = chn // npr

    mesh = plsc.VectorSubcoreMesh(
        core_axis_name="c", subcore_axis_name="s", num_cores=_NC, num_subcores=_NS
    )

    @functools.partial(
        pl.kernel,
        out_type=jax.ShapeDtypeStruct(out_shape, jnp.float32),
        mesh=mesh,
        compiler_params=pltpu.CompilerParams(use_tc_tiling_on_sc=False),
        scratch_types=[
            pltpu.VMEM((pt, _SEQ), jnp.int32),
            pltpu.VMEM((pt * _SEQ,), jnp.int32),
            pltpu.VMEM((chn * _SEQ, cw), jnp.int32),
            pltpu.VMEM((chn * _SEQ, cw), jnp.int32),
            pltpu.VMEM((och, 128 if pack_out else c), jnp.float32),
            pltpu.VMEM((och, 128 if pack_out else c), jnp.float32),
            pltpu.VMEM((c,), jnp.float32),
            pltpu.SemaphoreType.DMA,
            pltpu.SemaphoreType.DMA,
            pltpu.SemaphoreType.DMA,
            pltpu.SemaphoreType.DMA,
        ],
    )
    def body(table_hbm, idx_hbm, bias_hbm, out_hbm,
             idx2d, idxv, gbuf0, gbuf1, obuf0, obuf1, biasv,
             gsem0, gsem1, osem0, osem1):
        wid = lax.axis_index("s") * _NC + lax.axis_index("c")
        base = jnp.minimum(wid * pt, n - pt)

        pltpu.sync_copy(idx_hbm.at[pl.ds(base, pt), :], idx2d)
        pltpu.sync_copy(bias_hbm, biasv)

        lane = lax.broadcasted_iota(jnp.int32, (_L,), 0)

        @pl.loop(0, pt, unroll=4)
        def _flatten(i):
            v = idx2d[i, :]
            idxv[pl.ds(pl.multiple_of(i * _SEQ, _SEQ), _SEQ)] = v * _SEQ + lane

        bias_vecs = [biasv[pl.ds(j * _L, _L)] for j in range(cvec)]

        def fire(g, gbuf, gsem):
            for d in range(ndma):
                roff = pl.multiple_of(g * chn * _SEQ + d * rows, rows)
                pltpu.async_copy(
                    table_hbm.at[idxv.at[pl.ds(roff, rows)]],
                    gbuf.at[pl.ds(d * rows, rows)], gsem)

        def reduce_chunk(g, gbuf, gsem, obuf, osem):
            pltpu.make_async_copy(
                table_hbm.at[pl.ds(0, chn * _SEQ)], gbuf, gsem).wait()

            @pl.when(g >= 2)
            def _():
                pltpu.make_async_copy(out_hbm.at[pl.ds(0, och)], obuf, osem).wait()

            for nloc in range(chn):
                acc = list(bias_vecs)
                for s in range(_SEQ):
                    r = nloc * _SEQ + s
                    for b in range(cw // _L):
                        w = gbuf[r, pl.ds(b * _L, _L)]
                        lo = lax.bitcast_convert_type(
                            lax.shift_left(w, 16), jnp.float32)
                        hi = lax.bitcast_convert_type(w & _HIMASK, jnp.float32)
                        acc[2 * b] = acc[2 * b] + lo
                        acc[2 * b + 1] = acc[2 * b + 1] + hi
                for j in range(cvec):
                    v = acc[j]
                    if apply_elu:
                        v = jnp.where(v > 0.0, v, jnp.exp(v) - 1.0)
                    obuf[nloc // npr, pl.ds((nloc % npr) * c + j * _L, _L)] = v
            @pl.when(g + 2 < nchunk)
            def _():
                fire(g + 2, gbuf, gsem)
            pltpu.async_copy(
                obuf, out_hbm.at[pl.ds((base + g * chn) // npr, och)], osem)

        fire(0, gbuf0, gsem0)
        fire(1, gbuf1, gsem1)

        @pl.loop(0, nchunk // 2)
        def _pair(h):
            g0 = pl.multiple_of(h * 2, 2)
            reduce_chunk(g0, gbuf0, gsem0, obuf0, osem0)
            reduce_chunk(g0 + 1, gbuf1, gsem1, obuf1, osem1)

        pltpu.make_async_copy(out_hbm.at[pl.ds(0, och)], obuf0, osem0).wait()
        pltpu.make_async_copy(out_hbm.at[pl.ds(0, och)], obuf1, osem1).wait()

    return body


def _split_weights(W, c_in, c):
    wt = W.reshape(_SEQ, c_in, c).transpose(1, 0, 2)
    j = np.arange(c // 2)
    lo_ch = (j // _L) * 2 * _L + (j % _L)
    wa = wt[:, :, lo_ch].reshape(c_in, _SEQ * c // 2)
    wb = wt[:, :, lo_ch + _L].reshape(c_in, _SEQ * c // 2)
    return jnp.concatenate([wa, wb], axis=1)


def kernel(x, spiral_indices, W1, b1, W2, b2):
    n = x.shape[0]
    c0 = x.shape[1]
    c1 = W1.shape[1]
    c2 = W2.shape[1]

    blk = 2048
    grid1 = -(-n // blk)

    w1c = _split_weights(W1, c0, c1)
    w2c = _split_weights(W2, c1, c2)
    z = jnp.zeros_like(w2c)
    w2d = jnp.concatenate([
        jnp.concatenate([w2c, z], axis=1),
        jnp.concatenate([z, w2c], axis=1),
    ], axis=0)

    rr1 = _SEQ * c1 // 2 // 128
    y1 = _matmul_pack_tc(_mm1_body, x.reshape(n, c0), w1c, blk, rr1, grid1)
    g1 = _make_gather_reduce(n, c1, True, True, _CH)
    h1p = g1(y1.reshape(grid1 * blk * _SEQ, c1 // 2), spiral_indices, b1)

    blk2 = 1000
    rr2 = _SEQ * c2 // 128
    y2 = _matmul_pack_tc(_mm2_body, h1p, w2d, blk2, rr2, n // 2 // blk2)
    g2 = _make_gather_reduce(n, c2, False, True, 3 * _CH)
    out = g2(y2.reshape(n * _SEQ, c2 // 2), spiral_indices, b2)

    return out.reshape(n, c2)[:, :, None]
